# baseline (speedup 1.0000x reference)
import functools
import jax
import jax.numpy as jnp
from jax import lax
from jax.experimental import pallas as pl
from jax.experimental.pallas import tpu as pltpu
from jax.experimental.pallas import tpu_sc as plsc

_N = 512
_PAD = 160


def _sc_keep_kernel(idx_h_hbm, idx_w_hbm, keep_h_hbm, keep_w_hbm,
                    idx_v, ones_v, zeros_v, sem):
    c = lax.axis_index("c")
    s = lax.axis_index("s")
    wid = s * 2 + c

    def build(idx_hbm, out_hbm):
        ones = jnp.ones((16,), jnp.float32)
        zeros = jnp.zeros((16,), jnp.float32)
        for i in range(_N // 16):
            ones_v[pl.ds(i * 16, 16)] = ones
        for i in range(_PAD // 16):
            zeros_v[pl.ds(i * 16, 16)] = zeros
        pltpu.sync_copy(idx_hbm, idx_v)
        pltpu.sync_copy(ones_v, out_hbm)
        pltpu.async_copy(zeros_v, out_hbm.at[idx_v], sem).wait()

    @pl.when(wid == 0)
    def _():
        build(idx_h_hbm, keep_h_hbm)

    @pl.when(wid == 1)
    def _():
        build(idx_w_hbm, keep_w_hbm)


def _make_keeps(idx_h, idx_w):
    mesh = plsc.VectorSubcoreMesh(core_axis_name="c", subcore_axis_name="s")
    kern = functools.partial(
        pl.kernel,
        mesh=mesh,
        out_type=(
            jax.ShapeDtypeStruct((_N,), jnp.float32),
            jax.ShapeDtypeStruct((_N,), jnp.float32),
        ),
        scratch_types=[
            pltpu.VMEM((_PAD,), jnp.int32),
            pltpu.VMEM((_N,), jnp.float32),
            pltpu.VMEM((_PAD,), jnp.float32),
            pltpu.SemaphoreType.DMA,
        ],
    )(_sc_keep_kernel)
    return kern(idx_h, idx_w)


def _mul_kernel(kh_ref, kw_ref, x_ref, o_ref):
    mask = kh_ref[...] * kw_ref[...]
    o_ref[...] = x_ref[...] * mask[None, :, :]


def kernel(noised_image, cover_image, idx_H, idx_W):
    B, C, H, W = noised_image.shape
    x = noised_image.reshape(B * C, H, W)

    def _pad(idx):
        idx = idx.astype(jnp.int32)
        n = idx.shape[0]
        return jnp.concatenate([idx, jnp.broadcast_to(idx[:1], (_PAD - n,))])

    keep_h, keep_w = _make_keeps(_pad(idx_H), _pad(idx_W))

    G = 12
    out = pl.pallas_call(
        _mul_kernel,
        grid=(B * C // G,),
        in_specs=[
            pl.BlockSpec((H, 1), lambda i: (0, 0)),
            pl.BlockSpec((1, W), lambda i: (0, 0)),
            pl.BlockSpec((G, H, W), lambda i: (i, 0, 0)),
        ],
        out_specs=pl.BlockSpec((G, H, W), lambda i: (i, 0, 0)),
        out_shape=jax.ShapeDtypeStruct((B * C, H, W), x.dtype),
    )(keep_h.reshape(H, 1), keep_w.reshape(1, W), x)
    return (out.reshape(B, C, H, W), cover_image)


# speedup vs baseline: 1.0116x; 1.0116x over previous
import functools
import jax
import jax.numpy as jnp
from jax import lax
from jax.experimental import pallas as pl
from jax.experimental.pallas import tpu as pltpu
from jax.experimental.pallas import tpu_sc as plsc

_N = 512


def _sc_keep_kernel(idx_h_hbm, idx_w_hbm, ones_hbm, zeros_hbm,
                    keep_h_hbm, keep_w_hbm, idx_v, zeros_v, sem):
    c = lax.axis_index("c")
    s = lax.axis_index("s")
    wid = s * 2 + c

    def build(idx_hbm, out_hbm):
        pltpu.sync_copy(idx_hbm, idx_v)
        pltpu.sync_copy(zeros_hbm, zeros_v)
        pltpu.sync_copy(ones_hbm, out_hbm)
        pltpu.async_copy(zeros_v, out_hbm.at[idx_v], sem).wait()

    @pl.when(wid == 0)
    def _():
        build(idx_h_hbm, keep_h_hbm)

    @pl.when(wid == 1)
    def _():
        build(idx_w_hbm, keep_w_hbm)


def _make_keeps(idx_h, idx_w):
    n_idx = idx_h.shape[0]
    mesh = plsc.VectorSubcoreMesh(core_axis_name="c", subcore_axis_name="s")
    kern = functools.partial(
        pl.kernel,
        mesh=mesh,
        out_type=(
            jax.ShapeDtypeStruct((_N,), jnp.float32),
            jax.ShapeDtypeStruct((_N,), jnp.float32),
        ),
        scratch_types=[
            pltpu.VMEM((n_idx,), jnp.int32),
            pltpu.VMEM((n_idx,), jnp.float32),
            pltpu.SemaphoreType.DMA,
        ],
    )(_sc_keep_kernel)
    ones = jnp.ones((_N,), jnp.float32)
    zeros = jnp.zeros((n_idx,), jnp.float32)
    return kern(idx_h, idx_w, ones, zeros)


def _mul_kernel(kh_ref, kw_ref, x_ref, o_ref):
    mask = kh_ref[...] * kw_ref[...]
    o_ref[...] = x_ref[...] * mask[None, :, :]


def kernel(noised_image, cover_image, idx_H, idx_W):
    B, C, H, W = noised_image.shape
    x = noised_image.reshape(B * C, H, W)

    keep_h, keep_w = _make_keeps(idx_H.astype(jnp.int32), idx_W.astype(jnp.int32))

    G = 12
    out = pl.pallas_call(
        _mul_kernel,
        grid=(B * C // G,),
        in_specs=[
            pl.BlockSpec((H, 1), lambda i: (0, 0)),
            pl.BlockSpec((1, W), lambda i: (0, 0)),
            pl.BlockSpec((G, H, W), lambda i: (i, 0, 0)),
        ],
        out_specs=pl.BlockSpec((G, H, W), lambda i: (i, 0, 0)),
        out_shape=jax.ShapeDtypeStruct((B * C, H, W), x.dtype),
    )(keep_h.reshape(H, 1), keep_w.reshape(1, W), x)
    return (out.reshape(B, C, H, W), cover_image)


# speedup vs baseline: 1.0138x; 1.0022x over previous
import functools
import jax
import jax.numpy as jnp
from jax import lax
from jax.experimental import pallas as pl
from jax.experimental.pallas import tpu as pltpu
from jax.experimental.pallas import tpu_sc as plsc

_N = 512


def _sc_keep_kernel(idx_h_hbm, idx_w_hbm, ones_hbm, zeros_hbm,
                    keep_h_hbm, keep_w_hbm, idx_v, zeros_v, sem):
    c = lax.axis_index("c")
    s = lax.axis_index("s")

    def build(idx_hbm, out_hbm):
        pltpu.sync_copy(idx_hbm, idx_v)
        pltpu.sync_copy(zeros_hbm, zeros_v)
        pltpu.sync_copy(ones_hbm, out_hbm)
        pltpu.async_copy(zeros_v, out_hbm.at[idx_v], sem).wait()

    @pl.when((c == 0) & (s == 0))
    def _():
        build(idx_h_hbm, keep_h_hbm)

    @pl.when((c == 0) & (s == 1))
    def _():
        build(idx_w_hbm, keep_w_hbm)


def _make_keeps(idx_h, idx_w):
    n_idx = idx_h.shape[0]
    mesh = plsc.VectorSubcoreMesh(core_axis_name="c", subcore_axis_name="s")
    kern = functools.partial(
        pl.kernel,
        mesh=mesh,
        out_type=(
            jax.ShapeDtypeStruct((_N,), jnp.float32),
            jax.ShapeDtypeStruct((_N,), jnp.float32),
        ),
        scratch_types=[
            pltpu.VMEM((n_idx,), jnp.int32),
            pltpu.VMEM((n_idx,), jnp.float32),
            pltpu.SemaphoreType.DMA,
        ],
    )(_sc_keep_kernel)
    ones = jnp.ones((_N,), jnp.float32)
    zeros = jnp.zeros((n_idx,), jnp.float32)
    return kern(idx_h, idx_w, ones, zeros)


def _mul_kernel(kh_ref, kw_ref, x_ref, o_ref):
    mask = kh_ref[...] * kw_ref[...]
    o_ref[...] = x_ref[...] * mask[None, :, :]


def kernel(noised_image, cover_image, idx_H, idx_W):
    B, C, H, W = noised_image.shape
    x = noised_image.reshape(B * C, H, W)

    keep_h, keep_w = _make_keeps(idx_H.astype(jnp.int32), idx_W.astype(jnp.int32))

    G = 12
    out = pl.pallas_call(
        _mul_kernel,
        grid=(B * C // G,),
        in_specs=[
            pl.BlockSpec((H, 1), lambda i: (0, 0)),
            pl.BlockSpec((1, W), lambda i: (0, 0)),
            pl.BlockSpec((G, H, W), lambda i: (i, 0, 0)),
        ],
        out_specs=pl.BlockSpec((G, H, W), lambda i: (i, 0, 0)),
        out_shape=jax.ShapeDtypeStruct((B * C, H, W), x.dtype),
    )(keep_h.reshape(H, 1), keep_w.reshape(1, W), x)
    return (out.reshape(B, C, H, W), cover_image)


# speedup vs baseline: 1.0216x; 1.0077x over previous
import functools
import jax
import jax.numpy as jnp
from jax import lax
from jax.experimental import pallas as pl
from jax.experimental.pallas import tpu as pltpu
from jax.experimental.pallas import tpu_sc as plsc

_N = 512


def _sc_keep_kernel(idx_h_hbm, idx_w_hbm, ones_hbm, zeros_hbm,
                    keep_h_hbm, keep_w_hbm, idx_v, zeros_v, sem):
    c = lax.axis_index("c")
    s = lax.axis_index("s")

    def build(idx_hbm, out_hbm):
        a = pltpu.async_copy(idx_hbm, idx_v, sem)
        b = pltpu.async_copy(zeros_hbm, zeros_v, sem)
        d = pltpu.async_copy(ones_hbm, out_hbm, sem)
        a.wait()
        b.wait()
        d.wait()
        pltpu.async_copy(zeros_v, out_hbm.at[idx_v], sem).wait()

    @pl.when((c == 0) & (s == 0))
    def _():
        build(idx_h_hbm, keep_h_hbm)

    @pl.when((c == 0) & (s == 1))
    def _():
        build(idx_w_hbm, keep_w_hbm)


def _make_keeps(idx_h, idx_w):
    n_idx = idx_h.shape[0]
    mesh = plsc.VectorSubcoreMesh(core_axis_name="c", subcore_axis_name="s")
    kern = functools.partial(
        pl.kernel,
        mesh=mesh,
        out_type=(
            jax.ShapeDtypeStruct((_N,), jnp.float32),
            jax.ShapeDtypeStruct((_N,), jnp.float32),
        ),
        scratch_types=[
            pltpu.VMEM((n_idx,), jnp.int32),
            pltpu.VMEM((n_idx,), jnp.float32),
            pltpu.SemaphoreType.DMA,
        ],
    )(_sc_keep_kernel)
    ones = jnp.ones((_N,), jnp.float32)
    zeros = jnp.zeros((n_idx,), jnp.float32)
    return kern(idx_h, idx_w, ones, zeros)


def _mul_kernel(kh_ref, kw_ref, x_ref, o_ref):
    mask = kh_ref[...] * kw_ref[...]
    o_ref[...] = x_ref[...] * mask[None, :, :]


def kernel(noised_image, cover_image, idx_H, idx_W):
    B, C, H, W = noised_image.shape
    x = noised_image.reshape(B * C, H, W)

    keep_h, keep_w = _make_keeps(idx_H.astype(jnp.int32), idx_W.astype(jnp.int32))

    G = 12
    out = pl.pallas_call(
        _mul_kernel,
        grid=(B * C // G,),
        in_specs=[
            pl.BlockSpec((H, 1), lambda i: (0, 0)),
            pl.BlockSpec((1, W), lambda i: (0, 0)),
            pl.BlockSpec((G, H, W), lambda i: (i, 0, 0)),
        ],
        out_specs=pl.BlockSpec((G, H, W), lambda i: (i, 0, 0)),
        out_shape=jax.ShapeDtypeStruct((B * C, H, W), x.dtype),
    )(keep_h.reshape(H, 1), keep_w.reshape(1, W), x)
    return (out.reshape(B, C, H, W), cover_image)


# speedup vs baseline: 1.0226x; 1.0009x over previous
import functools
import jax
import jax.numpy as jnp
from jax import lax
from jax.experimental import pallas as pl
from jax.experimental.pallas import tpu as pltpu
from jax.experimental.pallas import tpu_sc as plsc


def _sc_keep_kernel(idx_h_hbm, idx_w_hbm, ones_h_hbm, ones_w_hbm,
                    zeros_h_hbm, zeros_w_hbm, keep_h_hbm, keep_w_hbm,
                    idx_h_v, idx_w_v, zeros_h_v, zeros_w_v, sem):
    c = lax.axis_index("c")
    s = lax.axis_index("s")

    def build(idx_hbm, ones_hbm, zeros_hbm, idx_v, zeros_v, out_hbm):
        a = pltpu.async_copy(idx_hbm, idx_v, sem)
        b = pltpu.async_copy(zeros_hbm, zeros_v, sem)
        d = pltpu.async_copy(ones_hbm, out_hbm, sem)
        a.wait()
        b.wait()
        d.wait()
        pltpu.async_copy(zeros_v, out_hbm.at[idx_v], sem).wait()

    @pl.when((c == 0) & (s == 0))
    def _():
        build(idx_h_hbm, ones_h_hbm, zeros_h_hbm, idx_h_v, zeros_h_v,
              keep_h_hbm)

    @pl.when((c == 0) & (s == 1))
    def _():
        build(idx_w_hbm, ones_w_hbm, zeros_w_hbm, idx_w_v, zeros_w_v,
              keep_w_hbm)


def _make_keeps(idx_h, idx_w, h, w):
    n_h = idx_h.shape[0]
    n_w = idx_w.shape[0]
    mesh = plsc.VectorSubcoreMesh(core_axis_name="c", subcore_axis_name="s")
    kern = functools.partial(
        pl.kernel,
        mesh=mesh,
        out_type=(
            jax.ShapeDtypeStruct((h,), jnp.float32),
            jax.ShapeDtypeStruct((w,), jnp.float32),
        ),
        scratch_types=[
            pltpu.VMEM((n_h,), jnp.int32),
            pltpu.VMEM((n_w,), jnp.int32),
            pltpu.VMEM((n_h,), jnp.float32),
            pltpu.VMEM((n_w,), jnp.float32),
            pltpu.SemaphoreType.DMA,
        ],
    )(_sc_keep_kernel)
    return kern(idx_h, idx_w,
                jnp.ones((h,), jnp.float32), jnp.ones((w,), jnp.float32),
                jnp.zeros((n_h,), jnp.float32), jnp.zeros((n_w,), jnp.float32))


def _mul_kernel(kh_ref, kw_ref, x_ref, o_ref):
    mask = kh_ref[...] * kw_ref[...]
    o_ref[...] = x_ref[...] * mask[None, :, :]


def kernel(noised_image, cover_image, idx_H, idx_W):
    B, C, H, W = noised_image.shape
    n = B * C
    x = noised_image.reshape(n, H, W)

    keep_h, keep_w = _make_keeps(idx_H.astype(jnp.int32),
                                 idx_W.astype(jnp.int32), H, W)

    G = 12 if n % 12 == 0 else (8 if n % 8 == 0 else 1)
    out = pl.pallas_call(
        _mul_kernel,
        grid=(n // G,),
        in_specs=[
            pl.BlockSpec((H, 1), lambda i: (0, 0)),
            pl.BlockSpec((1, W), lambda i: (0, 0)),
            pl.BlockSpec((G, H, W), lambda i: (i, 0, 0)),
        ],
        out_specs=pl.BlockSpec((G, H, W), lambda i: (i, 0, 0)),
        out_shape=jax.ShapeDtypeStruct((n, H, W), x.dtype),
    )(keep_h.reshape(H, 1), keep_w.reshape(1, W), x)
    return (out.reshape(B, C, H, W), cover_image)


# speedup vs baseline: 1.0364x; 1.0136x over previous
import functools
import jax
import jax.numpy as jnp
from jax import lax
from jax.experimental import pallas as pl
from jax.experimental.pallas import tpu as pltpu
from jax.experimental.pallas import tpu_sc as plsc


def _sc_keep_kernel(idx_h_hbm, idx_w_hbm, ones_h_hbm, ones_w_hbm,
                    zeros_h_hbm, zeros_w_hbm, keep_h_hbm, keep_w_hbm,
                    idx_h_v, idx_w_v, zeros_h_v, zeros_w_v, sem):
    c = lax.axis_index("c")
    s = lax.axis_index("s")

    def build(idx_hbm, ones_hbm, zeros_hbm, idx_v, zeros_v, out_hbm):
        a = pltpu.async_copy(idx_hbm, idx_v, sem)
        b = pltpu.async_copy(zeros_hbm, zeros_v, sem)
        d = pltpu.async_copy(ones_hbm, out_hbm, sem)
        a.wait()
        b.wait()
        d.wait()
        pltpu.async_copy(zeros_v, out_hbm.at[idx_v], sem).wait()

    @pl.when((c == 0) & (s == 0))
    def _():
        build(idx_h_hbm, ones_h_hbm, zeros_h_hbm, idx_h_v, zeros_h_v,
              keep_h_hbm)

    @pl.when((c == 0) & (s == 1))
    def _():
        build(idx_w_hbm, ones_w_hbm, zeros_w_hbm, idx_w_v, zeros_w_v,
              keep_w_hbm)


def _make_keeps(idx_h, idx_w, h, w):
    n_h = idx_h.shape[0]
    n_w = idx_w.shape[0]
    mesh = plsc.VectorSubcoreMesh(core_axis_name="c", subcore_axis_name="s",
                                  num_cores=1)
    kern = functools.partial(
        pl.kernel,
        mesh=mesh,
        out_type=(
            jax.ShapeDtypeStruct((h,), jnp.float32),
            jax.ShapeDtypeStruct((w,), jnp.float32),
        ),
        scratch_types=[
            pltpu.VMEM((n_h,), jnp.int32),
            pltpu.VMEM((n_w,), jnp.int32),
            pltpu.VMEM((n_h,), jnp.float32),
            pltpu.VMEM((n_w,), jnp.float32),
            pltpu.SemaphoreType.DMA,
        ],
    )(_sc_keep_kernel)
    return kern(idx_h, idx_w,
                jnp.ones((h,), jnp.float32), jnp.ones((w,), jnp.float32),
                jnp.zeros((n_h,), jnp.float32), jnp.zeros((n_w,), jnp.float32))


def _mul_kernel(kh_ref, kw_ref, x_ref, o_ref):
    mask = kh_ref[...] * kw_ref[...]
    o_ref[...] = x_ref[...] * mask[None, :, :]


def kernel(noised_image, cover_image, idx_H, idx_W):
    B, C, H, W = noised_image.shape
    n = B * C
    x = noised_image.reshape(n, H, W)

    keep_h, keep_w = _make_keeps(idx_H.astype(jnp.int32),
                                 idx_W.astype(jnp.int32), H, W)

    G = 12 if n % 12 == 0 else (8 if n % 8 == 0 else 1)
    out = pl.pallas_call(
        _mul_kernel,
        grid=(n // G,),
        in_specs=[
            pl.BlockSpec((H, 1), lambda i: (0, 0)),
            pl.BlockSpec((1, W), lambda i: (0, 0)),
            pl.BlockSpec((G, H, W), lambda i: (i, 0, 0)),
        ],
        out_specs=pl.BlockSpec((G, H, W), lambda i: (i, 0, 0)),
        out_shape=jax.ShapeDtypeStruct((n, H, W), x.dtype),
    )(keep_h.reshape(H, 1), keep_w.reshape(1, W), x)
    return (out.reshape(B, C, H, W), cover_image)
